# baseline (speedup 1.0000x reference)
import functools

import jax
import jax.numpy as jnp
from jax import lax
from jax.experimental import pallas as pl
from jax.experimental.pallas import tpu as pltpu
from jax.experimental.pallas import tpu_sc as plsc

B = 8
N = 1024
K = 20
EPS = 1e-5
BNK = float(B * N * K)
NW = 32
P = (B * N) // NW
CP = 128
F32 = jnp.float32
HI = jax.lax.Precision.HIGHEST


def _lrelu(v):
    return jnp.where(v >= 0, v, 0.2 * v)


def _bn_apply(v, mean, var, g, b):
    return (v - mean) / jnp.sqrt(var + EPS) * g + b


def _moments(hs_ref, hq_ref, cnt):
    s = jnp.sum(hs_ref[...], axis=0)
    q = jnp.sum(hq_ref[...], axis=0)
    mean = s / cnt
    var = q / cnt - mean * mean
    return mean, var


def _pad_cols(v, w):
    if v.shape[1] == w:
        return v
    return jnp.concatenate(
        [v, jnp.zeros((v.shape[0], w - v.shape[1]), F32)], axis=1)


def _topk_store(pair_ref, idx_ref, bofs):
    iota_n = lax.broadcasted_iota(jnp.int32, (N, N), 1)
    iota_k = lax.broadcasted_iota(jnp.int32, (N, K), 1)
    neg = jnp.float32(-jnp.inf)
    big = jnp.int32(1 << 30)

    m0 = jnp.max(pair_ref[...], axis=1, keepdims=True)

    def body(k, carry):
        m, idxacc = carry
        p = pair_ref[...]
        a = jnp.min(jnp.where(p == m, iota_n, big), axis=1,
                    keepdims=True)
        idxacc = jnp.where(iota_k == k, a + bofs, idxacc)
        pnew = jnp.where(iota_n == a, neg, p)
        pair_ref[...] = pnew
        mnew = jnp.max(pnew, axis=1, keepdims=True)
        return mnew, idxacc

    _, idxacc = lax.fori_loop(0, K, body, (m0, jnp.zeros((N, K), jnp.int32)))
    idx_ref[...] = idxacc


def _graph_common(xb, C, pair_ref, idx_ref, xpad_ref, b):
    dt = (((1,), (1,)), ((), ()))
    g = lax.dot_general(xb, xb, dt, preferred_element_type=F32)
    dd = lax.dot_general(jnp.ones((1, C), F32), xb * xb, dt, precision=HI,
                         preferred_element_type=F32)
    pair_ref[...] = 2.0 * g - dd
    xpad_ref[...] = _pad_cols(xb, CP)
    _topk_store(pair_ref, idx_ref, b * N)


def _ka1_body(x_ref, idx_ref, xpad_ref, pair_ref):
    _graph_common(x_ref[...], 3, pair_ref, idx_ref, xpad_ref,
                  pl.program_id(0))


def _ka_body(hmax_ref, hs_ref, hq_ref, g_ref, b_ref,
             idx_ref, xpad_ref, x_ref, pair_ref, C):
    b = pl.program_id(0)
    mean, var = _moments(hs_ref, hq_ref, BNK)
    xb = _lrelu(_bn_apply(hmax_ref[...], mean, var, g_ref[...], b_ref[...]))
    x_ref[...] = xb
    _graph_common(xb, C, pair_ref, idx_ref, xpad_ref, b)


def _make_ka1():
    return pl.pallas_call(
        _ka1_body,
        grid=(B,),
        in_specs=[pl.BlockSpec((N, 3), lambda b: (b, 0))],
        out_specs=[
            pl.BlockSpec((N, K), lambda b: (b, 0)),
            pl.BlockSpec((N, CP), lambda b: (b, 0)),
        ],
        out_shape=[
            jax.ShapeDtypeStruct((B * N, K), jnp.int32),
            jax.ShapeDtypeStruct((B * N, CP), F32),
        ],
        scratch_shapes=[pltpu.VMEM((N, N), F32)],
    )


def _make_ka(C):
    return pl.pallas_call(
        functools.partial(_ka_body, C=C),
        grid=(B,),
        in_specs=[
            pl.BlockSpec((N, C), lambda b: (b, 0)),
            pl.BlockSpec((B, 1, C), lambda b: (0, 0, 0)),
            pl.BlockSpec((B, 1, C), lambda b: (0, 0, 0)),
            pl.BlockSpec((1, C), lambda b: (0, 0)),
            pl.BlockSpec((1, C), lambda b: (0, 0)),
        ],
        out_specs=[
            pl.BlockSpec((N, K), lambda b: (b, 0)),
            pl.BlockSpec((N, CP), lambda b: (b, 0)),
            pl.BlockSpec((N, C), lambda b: (b, 0)),
        ],
        out_shape=[
            jax.ShapeDtypeStruct((B * N, K), jnp.int32),
            jax.ShapeDtypeStruct((B * N, CP), F32),
            jax.ShapeDtypeStruct((B * N, C), F32),
        ],
        scratch_shapes=[pltpu.VMEM((N, N), F32)],
    )


@functools.cache
def _make_sc_gather():
    PG = 4
    NG = P // PG
    mesh = plsc.VectorSubcoreMesh(core_axis_name="c", subcore_axis_name="s")

    NB = 4
    scratch = [pltpu.VMEM((P * K,), jnp.int32)]
    scratch += [pltpu.VMEM((PG * K, CP), F32) for _ in range(NB)]
    scratch += [pltpu.SemaphoreType.DMA for _ in range(2 * NB)]

    @functools.partial(
        pl.kernel,
        mesh=mesh,
        out_type=jax.ShapeDtypeStruct((B * N * K, CP), F32),
        scratch_types=scratch,
    )
    def sc_fn(idx_hbm, xpad_hbm, feat_hbm, idx_v, *bufsem):
        bufs = bufsem[:NB]
        sgs = bufsem[NB:2 * NB]
        sws = bufsem[2 * NB:]
        wid = lax.axis_index("s") * 2 + lax.axis_index("c")
        base = wid * P
        pltpu.sync_copy(idx_hbm.at[pl.ds(base * K, P * K)], idx_v)

        def gstart(gi, b):
            pltpu.async_copy(
                xpad_hbm.at[idx_v.at[pl.ds(gi * PG * K, PG * K)]],
                bufs[b], sgs[b])

        for b in range(NB):
            gstart(b, b)

        def outer(go, _):
            for b in range(NB):
                gi = go * NB + b
                pltpu.make_async_copy(
                    xpad_hbm.at[idx_v.at[pl.ds(gi * PG * K, PG * K)]],
                    bufs[b], sgs[b]).wait()
                pltpu.async_copy(
                    bufs[b],
                    feat_hbm.at[pl.ds((base + gi * PG) * K, PG * K)],
                    sws[b])

                @pl.when(gi + NB < NG)
                def _():
                    pltpu.make_async_copy(
                        bufs[b],
                        feat_hbm.at[pl.ds((base + gi * PG) * K, PG * K)],
                        sws[b]).wait()
                    gstart(gi + NB, b)
            return 0
        lax.fori_loop(0, NG // NB, outer, 0)
        for b in range(NB):
            gi = NG - NB + b
            pltpu.make_async_copy(
                bufs[b],
                feat_hbm.at[pl.ds((base + gi * PG) * K, PG * K)],
                sws[b]).wait()

    return sc_fn


def _kb_body(diff_ref, x_ref, wt_ref, hmax_ref, hs_ref, hq_ref, C, O, T):
    NT = N // T
    acc = (jnp.zeros((1, O), F32), jnp.zeros((1, O), F32))

    def sub(t, carry):
        acc_s, acc_q = carry
        fblk = diff_ref[pl.ds(t * NT * K, NT * K), :C]
        xb = x_ref[pl.ds(t * NT, NT), :]
        ctr = jnp.broadcast_to(xb[:, None, :], (NT, K, C)).reshape(NT * K, C)
        f2 = jnp.concatenate([fblk - ctr, ctr], axis=1)
        h = jnp.dot(f2, wt_ref[...], preferred_element_type=F32)
        hmax_ref[pl.ds(t * NT, NT), :] = jnp.max(
            h.reshape(NT, K, O), axis=1)
        ones = jnp.ones((1, NT * K), F32)
        return (acc_s + jnp.dot(ones, h, precision=HI,
                                preferred_element_type=F32),
                acc_q + jnp.dot(ones, h * h, precision=HI,
                                preferred_element_type=F32))

    acc_s, acc_q = lax.fori_loop(0, T, sub, acc)
    hs_ref[...] = acc_s[None]
    hq_ref[...] = acc_q[None]


def _make_kb(C, O, T=4):
    return pl.pallas_call(
        functools.partial(_kb_body, C=C, O=O, T=T),
        grid=(B,),
        in_specs=[
            pl.BlockSpec((N * K, CP), lambda b: (b, 0)),
            pl.BlockSpec((N, C), lambda b: (b, 0)),
            pl.BlockSpec((2 * C, O), lambda b: (0, 0)),
        ],
        out_specs=[
            pl.BlockSpec((N, O), lambda b: (b, 0)),
            pl.BlockSpec((1, 1, O), lambda b: (b, 0, 0)),
            pl.BlockSpec((1, 1, O), lambda b: (b, 0, 0)),
        ],
        out_shape=[
            jax.ShapeDtypeStruct((B * N, O), F32),
            jax.ShapeDtypeStruct((B, 1, O), F32),
            jax.ShapeDtypeStruct((B, 1, O), F32),
        ],
    )


def _final_body(x1_ref, x2_ref, x3_ref, hmax_ref, hs_ref, hq_ref,
                g4_ref, b4_ref, w5_ref, g5_ref, b5_ref,
                l1o_ref, g6o_ref, b6o_ref, l2o_ref, l2ob_ref, g7o_ref,
                b7o_ref, l3o_ref, l3ob_ref,
                l1n_ref, g6n_ref, b6n_ref, l2n_ref, l2nb_ref, g7n_ref,
                b7n_ref, l3n_ref, l3nb_ref,
                old_ref, new_ref, feat_s):
    mean4, var4 = _moments(hs_ref, hq_ref, BNK)

    def h5_of(b):
        rows = pl.ds(b * N, N)
        x4b = _lrelu(_bn_apply(hmax_ref[rows, :], mean4, var4,
                               g4_ref[...], b4_ref[...]))
        xcb = jnp.concatenate(
            [x1_ref[rows, :], x2_ref[rows, :], x3_ref[rows, :], x4b], axis=1)
        return jnp.dot(xcb, w5_ref[...], preferred_element_type=F32)

    def pass1(b, carry):
        acc_s, acc_q = carry
        h5b = h5_of(b)
        return (acc_s + jnp.sum(h5b, axis=0, keepdims=True),
                acc_q + jnp.sum(h5b * h5b, axis=0, keepdims=True))

    acc_s, acc_q = lax.fori_loop(
        0, B, pass1,
        (jnp.zeros((1, 1024), F32), jnp.zeros((1, 1024), F32)))
    cnt = float(B * N)
    mean5 = acc_s / cnt
    var5 = acc_q / cnt - mean5 * mean5

    def pass2(b, _):
        hb = _lrelu(_bn_apply(h5_of(b), mean5, var5,
                              g5_ref[...], b5_ref[...]))
        fmax = jnp.max(hb, axis=0, keepdims=True)
        fmean = jnp.sum(hb, axis=0, keepdims=True) * (1.0 / N)
        feat_s[pl.ds(b, 1), :] = jnp.concatenate([fmax, fmean], axis=1)
        return 0
    lax.fori_loop(0, B, pass2, 0)

    feat = feat_s[...]

    def head(l1, g6, b6, l2, l2b, g7, b7, l3, l3b):
        h = jnp.dot(feat, l1[...], preferred_element_type=F32)
        m = jnp.sum(h, axis=0, keepdims=True) / B
        v = jnp.sum(h * h, axis=0, keepdims=True) / B - m * m
        h = _lrelu(_bn_apply(h, m, v, g6[...], b6[...]))
        h = jnp.dot(h, l2[...], preferred_element_type=F32) + l2b[...]
        m = jnp.sum(h, axis=0, keepdims=True) / B
        v = jnp.sum(h * h, axis=0, keepdims=True) / B - m * m
        h = _lrelu(_bn_apply(h, m, v, g7[...], b7[...]))
        return jnp.dot(h, l3[...], preferred_element_type=F32) + l3b[...]

    old_ref[...] = head(l1o_ref, g6o_ref, b6o_ref, l2o_ref, l2ob_ref,
                        g7o_ref, b7o_ref, l3o_ref, l3ob_ref)
    new_ref[...] = head(l1n_ref, g6n_ref, b6n_ref, l2n_ref, l2nb_ref,
                        g7n_ref, b7n_ref, l3n_ref, l3nb_ref)


_final_call = pl.pallas_call(
    _final_body,
    out_shape=[
        jax.ShapeDtypeStruct((B, 10), F32),
        jax.ShapeDtypeStruct((B, 10), F32),
    ],
    scratch_shapes=[
        pltpu.VMEM((B, 2048), F32),
    ],
)

_ka1_call = _make_ka1()
_ka2_call = _make_ka(64)
_ka3_call = _make_ka(64)
_ka4_call = _make_ka(128)
_kb1_call = _make_kb(3, 64)
_kb2_call = _make_kb(64, 64)
_kb3_call = _make_kb(64, 128)
_kb4_call = _make_kb(128, 256)


def kernel(x, W1, g1, b1, W2, g2, b2, W3, g3, b3, W4, g4, b4, W5, g5, b5,
           L1o, g6o, b6o, L2ow, L2ob, g7o, b7o, L3ow, L3ob,
           L1n, g6n, b6n, L2nw, L2nb, g7n, b7n, L3nw, L3nb):
    r = lambda a: a.reshape(1, -1)
    xt0 = x.transpose(0, 2, 1).reshape(B * N, 3)

    idx, xpad = _ka1_call(xt0)
    diff = _make_sc_gather()(idx.reshape(-1), xpad)
    hm1, hs1, hq1 = _kb1_call(diff, xt0, W1.T)

    idx, xpad, x1 = _ka2_call(hm1, hs1, hq1, r(g1), r(b1))
    diff = _make_sc_gather()(idx.reshape(-1), xpad)
    hm2, hs2, hq2 = _kb2_call(diff, x1, W2.T)

    idx, xpad, x2 = _ka3_call(hm2, hs2, hq2, r(g2), r(b2))
    diff = _make_sc_gather()(idx.reshape(-1), xpad)
    hm3, hs3, hq3 = _kb3_call(diff, x2, W3.T)

    idx, xpad, x3 = _ka4_call(hm3, hs3, hq3, r(g3), r(b3))
    diff = _make_sc_gather()(idx.reshape(-1), xpad)
    hm4, hs4, hq4 = _kb4_call(diff, x3, W4.T)

    old, new = _final_call(
        x1, x2, x3, hm4, hs4, hq4, r(g4), r(b4),
        W5.T, r(g5), r(b5),
        L1o.T, r(g6o), r(b6o), L2ow.T, r(L2ob), r(g7o), r(b7o),
        L3ow.T, r(L3ob),
        L1n.T, r(g6n), r(b6n), L2nw.T, r(L2nb), r(g7n), r(b7n),
        L3nw.T, r(L3nb))
    return (old, new)


# speedup vs baseline: 1.2271x; 1.2271x over previous
import functools

import jax
import jax.numpy as jnp
from jax import lax
from jax.experimental import pallas as pl
from jax.experimental.pallas import tpu as pltpu
from jax.experimental.pallas import tpu_sc as plsc

B = 8
N = 1024
K = 20
EPS = 1e-5
BNK = float(B * N * K)
NW = 32
P = (B * N) // NW
CP = 128
F32 = jnp.float32
HI = jax.lax.Precision.HIGHEST


def _lrelu(v):
    return jnp.where(v >= 0, v, 0.2 * v)


def _bn_apply(v, mean, var, g, b):
    return (v - mean) / jnp.sqrt(var + EPS) * g + b


def _moments(hs_ref, hq_ref, cnt):
    s = jnp.sum(hs_ref[...], axis=0)
    q = jnp.sum(hq_ref[...], axis=0)
    mean = s / cnt
    var = q / cnt - mean * mean
    return mean, var


def _pad_cols(v, w):
    if v.shape[1] == w:
        return v
    return jnp.concatenate(
        [v, jnp.zeros((v.shape[0], w - v.shape[1]), F32)], axis=1)


def _topk_store(pair_ref, idx_ref, bofs):
    iota_n = lax.broadcasted_iota(jnp.int32, (N, N), 1)
    iota_k = lax.broadcasted_iota(jnp.int32, (N, K), 1)
    neg = jnp.float32(-jnp.inf)
    big = jnp.int32(1 << 30)

    m0 = jnp.max(pair_ref[...], axis=1, keepdims=True)

    def body(k, carry):
        m, idxacc = carry
        p = pair_ref[...]
        a = jnp.min(jnp.where(p == m, iota_n, big), axis=1,
                    keepdims=True)
        idxacc = jnp.where(iota_k == k, a + bofs, idxacc)
        pnew = jnp.where(iota_n == a, neg, p)
        pair_ref[...] = pnew
        mnew = jnp.max(pnew, axis=1, keepdims=True)
        return mnew, idxacc

    _, idxacc = lax.fori_loop(0, K, body, (m0, jnp.zeros((N, K), jnp.int32)))
    idx_ref[...] = idxacc


def _graph_common(xb, C, pair_ref, idx_ref, xpad_ref, b):
    dt = (((1,), (1,)), ((), ()))
    g = lax.dot_general(xb, xb, dt, preferred_element_type=F32)
    dd = lax.dot_general(jnp.ones((1, C), F32), xb * xb, dt, precision=HI,
                         preferred_element_type=F32)
    pair_ref[...] = 2.0 * g - dd
    xpad_ref[...] = _pad_cols(xb, CP)
    _topk_store(pair_ref, idx_ref, b * N)


def _ka1_body(x_ref, idx_ref, xpad_ref, pair_ref):
    _graph_common(x_ref[...], 3, pair_ref, idx_ref, xpad_ref,
                  pl.program_id(0))


def _ka_body(hmax_ref, hs_ref, hq_ref, g_ref, b_ref,
             idx_ref, xpad_ref, x_ref, pair_ref, C):
    b = pl.program_id(0)
    mean, var = _moments(hs_ref, hq_ref, BNK)
    xb = _lrelu(_bn_apply(hmax_ref[...], mean, var, g_ref[...], b_ref[...]))
    x_ref[...] = xb
    _graph_common(xb, C, pair_ref, idx_ref, xpad_ref, b)


def _make_ka1():
    return pl.pallas_call(
        _ka1_body,
        grid=(B,),
        in_specs=[pl.BlockSpec((N, 3), lambda b: (b, 0))],
        out_specs=[
            pl.BlockSpec((N, K), lambda b: (b, 0)),
            pl.BlockSpec((N, CP), lambda b: (b, 0)),
        ],
        out_shape=[
            jax.ShapeDtypeStruct((B * N, K), jnp.int32),
            jax.ShapeDtypeStruct((B * N, CP), F32),
        ],
        scratch_shapes=[pltpu.VMEM((N, N), F32)],
    )


def _make_ka(C):
    return pl.pallas_call(
        functools.partial(_ka_body, C=C),
        grid=(B,),
        in_specs=[
            pl.BlockSpec((N, C), lambda b: (b, 0)),
            pl.BlockSpec((B, 1, C), lambda b: (0, 0, 0)),
            pl.BlockSpec((B, 1, C), lambda b: (0, 0, 0)),
            pl.BlockSpec((1, C), lambda b: (0, 0)),
            pl.BlockSpec((1, C), lambda b: (0, 0)),
        ],
        out_specs=[
            pl.BlockSpec((N, K), lambda b: (b, 0)),
            pl.BlockSpec((N, CP), lambda b: (b, 0)),
            pl.BlockSpec((N, C), lambda b: (b, 0)),
        ],
        out_shape=[
            jax.ShapeDtypeStruct((B * N, K), jnp.int32),
            jax.ShapeDtypeStruct((B * N, CP), F32),
            jax.ShapeDtypeStruct((B * N, C), F32),
        ],
        scratch_shapes=[pltpu.VMEM((N, N), F32)],
    )


@functools.cache
def _make_sc_gather():
    PG = 4
    NG = P // PG
    mesh = plsc.VectorSubcoreMesh(core_axis_name="c", subcore_axis_name="s")

    NB = 4
    scratch = [pltpu.VMEM((P * K,), jnp.int32)]
    scratch += [pltpu.VMEM((PG * K, CP), F32) for _ in range(NB)]
    scratch += [pltpu.SemaphoreType.DMA for _ in range(2 * NB)]

    @functools.partial(
        pl.kernel,
        mesh=mesh,
        out_type=jax.ShapeDtypeStruct((B * N * K, CP), F32),
        scratch_types=scratch,
    )
    def sc_fn(idx_hbm, xpad_hbm, feat_hbm, idx_v, *bufsem):
        bufs = bufsem[:NB]
        sgs = bufsem[NB:2 * NB]
        sws = bufsem[2 * NB:]
        wid = lax.axis_index("s") * 2 + lax.axis_index("c")
        base = wid * P
        pltpu.sync_copy(idx_hbm.at[pl.ds(base * K, P * K)], idx_v)

        def gstart(gi, b):
            pltpu.async_copy(
                xpad_hbm.at[idx_v.at[pl.ds(gi * PG * K, PG * K)]],
                bufs[b], sgs[b])

        for b in range(NB):
            gstart(b, b)

        def outer(go, _):
            for b in range(NB):
                gi = go * NB + b
                pltpu.make_async_copy(
                    xpad_hbm.at[idx_v.at[pl.ds(gi * PG * K, PG * K)]],
                    bufs[b], sgs[b]).wait()
                pltpu.async_copy(
                    bufs[b],
                    feat_hbm.at[pl.ds((base + gi * PG) * K, PG * K)],
                    sws[b])

                @pl.when(gi + NB < NG)
                def _():
                    pltpu.make_async_copy(
                        bufs[b],
                        feat_hbm.at[pl.ds((base + gi * PG) * K, PG * K)],
                        sws[b]).wait()
                    gstart(gi + NB, b)
            return 0
        lax.fori_loop(0, NG // NB, outer, 0)
        for b in range(NB):
            gi = NG - NB + b
            pltpu.make_async_copy(
                bufs[b],
                feat_hbm.at[pl.ds((base + gi * PG) * K, PG * K)],
                sws[b]).wait()

    return sc_fn


def _kb_body(diff_ref, x_ref, wt_ref, hmax_ref, hs_ref, hq_ref, C, O, T):
    NT = N // T
    acc = (jnp.zeros((1, O), F32), jnp.zeros((1, O), F32))

    def sub(t, carry):
        acc_s, acc_q = carry
        fblk = diff_ref[pl.ds(t * NT * K, NT * K), :C]
        xb = x_ref[pl.ds(t * NT, NT), :]
        ctr = jnp.broadcast_to(xb[:, None, :], (NT, K, C)).reshape(NT * K, C)
        f2 = jnp.concatenate([fblk - ctr, ctr], axis=1)
        h = jnp.dot(f2, wt_ref[...], preferred_element_type=F32)
        hmax_ref[pl.ds(t * NT, NT), :] = jnp.max(
            h.reshape(NT, K, O), axis=1)
        return (acc_s + jnp.sum(h, axis=0, keepdims=True),
                acc_q + jnp.sum(h * h, axis=0, keepdims=True))

    acc_s, acc_q = lax.fori_loop(0, T, sub, acc)
    hs_ref[...] = acc_s[None]
    hq_ref[...] = acc_q[None]


def _make_kb(C, O, T=4):
    return pl.pallas_call(
        functools.partial(_kb_body, C=C, O=O, T=T),
        grid=(B,),
        in_specs=[
            pl.BlockSpec((N * K, CP), lambda b: (b, 0)),
            pl.BlockSpec((N, C), lambda b: (b, 0)),
            pl.BlockSpec((2 * C, O), lambda b: (0, 0)),
        ],
        out_specs=[
            pl.BlockSpec((N, O), lambda b: (b, 0)),
            pl.BlockSpec((1, 1, O), lambda b: (b, 0, 0)),
            pl.BlockSpec((1, 1, O), lambda b: (b, 0, 0)),
        ],
        out_shape=[
            jax.ShapeDtypeStruct((B * N, O), F32),
            jax.ShapeDtypeStruct((B, 1, O), F32),
            jax.ShapeDtypeStruct((B, 1, O), F32),
        ],
    )


def _final_body(x1_ref, x2_ref, x3_ref, hmax_ref, hs_ref, hq_ref,
                g4_ref, b4_ref, w5_ref, g5_ref, b5_ref,
                l1o_ref, g6o_ref, b6o_ref, l2o_ref, l2ob_ref, g7o_ref,
                b7o_ref, l3o_ref, l3ob_ref,
                l1n_ref, g6n_ref, b6n_ref, l2n_ref, l2nb_ref, g7n_ref,
                b7n_ref, l3n_ref, l3nb_ref,
                old_ref, new_ref, feat_s):
    mean4, var4 = _moments(hs_ref, hq_ref, BNK)

    def h5_of(b):
        rows = pl.ds(b * N, N)
        x4b = _lrelu(_bn_apply(hmax_ref[rows, :], mean4, var4,
                               g4_ref[...], b4_ref[...]))
        xcb = jnp.concatenate(
            [x1_ref[rows, :], x2_ref[rows, :], x3_ref[rows, :], x4b], axis=1)
        return jnp.dot(xcb, w5_ref[...], preferred_element_type=F32)

    def pass1(b, carry):
        acc_s, acc_q = carry
        h5b = h5_of(b)
        return (acc_s + jnp.sum(h5b, axis=0, keepdims=True),
                acc_q + jnp.sum(h5b * h5b, axis=0, keepdims=True))

    acc_s, acc_q = lax.fori_loop(
        0, B, pass1,
        (jnp.zeros((1, 1024), F32), jnp.zeros((1, 1024), F32)))
    cnt = float(B * N)
    mean5 = acc_s / cnt
    var5 = acc_q / cnt - mean5 * mean5

    def pass2(b, _):
        hb = _lrelu(_bn_apply(h5_of(b), mean5, var5,
                              g5_ref[...], b5_ref[...]))
        fmax = jnp.max(hb, axis=0, keepdims=True)
        fmean = jnp.sum(hb, axis=0, keepdims=True) * (1.0 / N)
        feat_s[pl.ds(b, 1), :] = jnp.concatenate([fmax, fmean], axis=1)
        return 0
    lax.fori_loop(0, B, pass2, 0)

    feat = feat_s[...]

    def head(l1, g6, b6, l2, l2b, g7, b7, l3, l3b):
        h = jnp.dot(feat, l1[...], preferred_element_type=F32)
        m = jnp.sum(h, axis=0, keepdims=True) / B
        v = jnp.sum(h * h, axis=0, keepdims=True) / B - m * m
        h = _lrelu(_bn_apply(h, m, v, g6[...], b6[...]))
        h = jnp.dot(h, l2[...], preferred_element_type=F32) + l2b[...]
        m = jnp.sum(h, axis=0, keepdims=True) / B
        v = jnp.sum(h * h, axis=0, keepdims=True) / B - m * m
        h = _lrelu(_bn_apply(h, m, v, g7[...], b7[...]))
        return jnp.dot(h, l3[...], preferred_element_type=F32) + l3b[...]

    old_ref[...] = head(l1o_ref, g6o_ref, b6o_ref, l2o_ref, l2ob_ref,
                        g7o_ref, b7o_ref, l3o_ref, l3ob_ref)
    new_ref[...] = head(l1n_ref, g6n_ref, b6n_ref, l2n_ref, l2nb_ref,
                        g7n_ref, b7n_ref, l3n_ref, l3nb_ref)


_final_call = pl.pallas_call(
    _final_body,
    out_shape=[
        jax.ShapeDtypeStruct((B, 10), F32),
        jax.ShapeDtypeStruct((B, 10), F32),
    ],
    scratch_shapes=[
        pltpu.VMEM((B, 2048), F32),
    ],
)

_ka1_call = _make_ka1()
_ka2_call = _make_ka(64)
_ka3_call = _make_ka(64)
_ka4_call = _make_ka(128)
_kb1_call = _make_kb(3, 64)
_kb2_call = _make_kb(64, 64)
_kb3_call = _make_kb(64, 128)
_kb4_call = _make_kb(128, 256)


def kernel(x, W1, g1, b1, W2, g2, b2, W3, g3, b3, W4, g4, b4, W5, g5, b5,
           L1o, g6o, b6o, L2ow, L2ob, g7o, b7o, L3ow, L3ob,
           L1n, g6n, b6n, L2nw, L2nb, g7n, b7n, L3nw, L3nb):
    r = lambda a: a.reshape(1, -1)
    xt0 = x.transpose(0, 2, 1).reshape(B * N, 3)

    idx, xpad = _ka1_call(xt0)
    diff = _make_sc_gather()(idx.reshape(-1), xpad)
    hm1, hs1, hq1 = _kb1_call(diff, xt0, W1.T)

    idx, xpad, x1 = _ka2_call(hm1, hs1, hq1, r(g1), r(b1))
    diff = _make_sc_gather()(idx.reshape(-1), xpad)
    hm2, hs2, hq2 = _kb2_call(diff, x1, W2.T)

    idx, xpad, x2 = _ka3_call(hm2, hs2, hq2, r(g2), r(b2))
    diff = _make_sc_gather()(idx.reshape(-1), xpad)
    hm3, hs3, hq3 = _kb3_call(diff, x2, W3.T)

    idx, xpad, x3 = _ka4_call(hm3, hs3, hq3, r(g3), r(b3))
    diff = _make_sc_gather()(idx.reshape(-1), xpad)
    hm4, hs4, hq4 = _kb4_call(diff, x3, W4.T)

    old, new = _final_call(
        x1, x2, x3, hm4, hs4, hq4, r(g4), r(b4),
        W5.T, r(g5), r(b5),
        L1o.T, r(g6o), r(b6o), L2ow.T, r(L2ob), r(g7o), r(b7o),
        L3ow.T, r(L3ob),
        L1n.T, r(g6n), r(b6n), L2nw.T, r(L2nb), r(g7n), r(b7n),
        L3nw.T, r(L3nb))
    return (old, new)
